# baseline (speedup 1.0000x reference)
import functools

import jax
import jax.numpy as jnp
from jax import lax
from jax.experimental import pallas as pl
from jax.experimental.pallas import tpu as pltpu
from jax.experimental.pallas import tpu_sc as plsc

N = 10000
NP = 10240
E = 320000
EP = 327680
EB = EP // 128
D_IN = 128
H = 528
HP = 640
FC = 128
NCH = 5
B = 8
BN = 512
GRID_N = NP // BN
TILES = 16
RPT = NP // TILES
WBLK = EB // 32
_f32 = jnp.float32


def _sc_mesh():
    return plsc.VectorSubcoreMesh(core_axis_name="c", subcore_axis_name="s")


def _zero_fill(zeros_v, acc, sid):
    nz = zeros_v.shape[0]

    @pl.loop(0, RPT // nz)
    def _(r):
        pltpu.sync_copy(zeros_v, acc.at[pl.ds(sid * RPT + r * nz, nz)])


def _fill_zeros_buf(zeros_v):
    nr, nc = zeros_v.shape

    @pl.loop(0, nr)
    def _(r):
        @pl.loop(0, nc // 16)
        def _(q):
            zeros_v[r, pl.ds(q * 16, 16)] = jnp.zeros((16,), _f32)


def _sc_degree(dst2d):
    @functools.partial(
        pl.kernel,
        mesh=_sc_mesh(),
        out_type=jax.ShapeDtypeStruct((2 * NP, FC), _f32),
        scratch_types=[
            pltpu.VMEM((WBLK, 128), jnp.int32),
            pltpu.VMEM((128, FC), _f32),
            pltpu.VMEM((64, FC), _f32),
            pltpu.VMEM_SHARED((NP, FC), _f32),
        ],
    )
    def deg_kernel(dst_hbm, out_hbm, idx_v, ones_v, zeros_v, acc):
        cid = lax.axis_index("c")
        sid = lax.axis_index("s")
        wid = cid * TILES + sid

        @pl.loop(0, 128)
        def _(r):
            @pl.loop(0, FC // 16)
            def _(q):
                ones_v[r, pl.ds(q * 16, 16)] = jnp.ones((16,), _f32)

        @pl.loop(0, 64)
        def _(r):
            @pl.loop(0, FC // 16)
            def _(q):
                zeros_v[r, pl.ds(q * 16, 16)] = jnp.zeros((16,), _f32)

        _zero_fill(zeros_v, acc, sid)
        plsc.subcore_barrier()

        pltpu.sync_copy(dst_hbm.at[pl.ds(wid * WBLK, WBLK)], idx_v)

        @pl.loop(0, WBLK)
        def _(j):
            pltpu.sync_copy(ones_v, acc.at[idx_v.at[j]], add=True)

        plsc.subcore_barrier()
        pltpu.sync_copy(
            acc.at[pl.ds(sid * RPT, RPT)],
            out_hbm.at[pl.ds(cid * NP + sid * RPT, RPT)],
        )

    return deg_kernel(dst2d)


def _sc_propagate(src2d, dst2d, hs):
    out_t = tuple(jax.ShapeDtypeStruct((2 * NP, FC), _f32)
                  for _ in range(NCH))

    NBUF = 2
    INFL = NBUF - 1
    BR = 128
    RPW = EP // BR // 32
    NST = 2
    SB = RPW // NST

    @functools.partial(
        pl.kernel,
        mesh=_sc_mesh(),
        out_type=out_t,
        scratch_types=[
            pltpu.VMEM((SB, BR), jnp.int32),
            pltpu.VMEM((SB, BR), jnp.int32),
        ]
        + [pltpu.VMEM((BR, FC), _f32) for _ in range(NBUF)]
        + [pltpu.SemaphoreType.DMA for _ in range(NBUF)]
        + [pltpu.VMEM_SHARED((NP, FC), _f32)],
    )
    def prop_kernel(src_hbm, dst_hbm, h0, h1, h2, h3, h4,
                    s0, s1, s2, s3, s4, sidx_v, didx_v, *rest):
        bufs = rest[:NBUF]
        sems = rest[NBUF:2 * NBUF]
        acc = rest[2 * NBUF]
        cid = lax.axis_index("c")
        sid = lax.axis_index("s")
        wid = cid * TILES + sid

        def run_chunk(h_hbm, s_hbm):
            with jax.named_scope("zero_fill"):
                _fill_zeros_buf(bufs[0])

                @pl.loop(0, RPT // BR)
                def _(r):
                    pltpu.sync_copy(bufs[0],
                                    acc.at[pl.ds(sid * RPT + r * BR, BR)])

                plsc.subcore_barrier()

            for st in range(NST):
                with jax.named_scope("idx_load"):
                    base = wid * RPW + st * SB
                    pltpu.sync_copy(src_hbm.at[pl.ds(base, SB)], sidx_v)
                    pltpu.sync_copy(dst_hbm.at[pl.ds(base, SB)], didx_v)

                with jax.named_scope("edge_pipeline"):
                    for b in range(INFL):
                        pltpu.async_copy(h_hbm.at[sidx_v.at[b]],
                                         bufs[b], sems[b])

                    @pl.loop(0, SB // NBUF)
                    def _(q):
                        j0 = NBUF * q
                        for b in range(NBUF):
                            j = j0 + b
                            pltpu.make_async_copy(
                                h_hbm.at[sidx_v.at[j]],
                                bufs[b], sems[b]).wait()
                            pltpu.sync_copy(bufs[b], acc.at[didx_v.at[j]],
                                            add=True)
                            jn = lax.min(j + INFL, SB - 1)
                            bi = (b + INFL) % NBUF
                            pltpu.async_copy(h_hbm.at[sidx_v.at[jn]],
                                             bufs[bi], sems[bi])

                    for b in range(INFL):
                        pltpu.make_async_copy(
                            h_hbm.at[sidx_v.at[SB - 1]],
                            bufs[b], sems[b]).wait()

            with jax.named_scope("writeback"):
                plsc.subcore_barrier()
                pltpu.sync_copy(acc.at[pl.ds(sid * RPT, RPT)],
                                s_hbm.at[pl.ds(cid * NP + sid * RPT, RPT)])
                plsc.subcore_barrier()

        for h_hbm, s_hbm in zip((h0, h1, h2, h3, h4), (s0, s1, s2, s3, s4)):
            run_chunk(h_hbm, s_hbm)

    return prop_kernel(src2d, dst2d, *hs)


def _chunk_in_specs():
    specs = []
    for _ in range(NCH):
        specs.append(pl.BlockSpec((BN, FC), lambda i: (i, 0)))
        specs.append(pl.BlockSpec((BN, FC), lambda i: (i + GRID_N, 0)))
    return specs


def _tc_matmul1(x_pad, w1p):
    def body(x_ref, w_ref, *orefs):
        m = jnp.dot(x_ref[...], w_ref[...], preferred_element_type=_f32)
        for c, ref in enumerate(orefs):
            ref[...] = m[:, c * FC:(c + 1) * FC]

    return pl.pallas_call(
        body,
        grid=(GRID_N,),
        in_specs=[
            pl.BlockSpec((BN, D_IN), lambda i: (i, 0)),
            pl.BlockSpec((D_IN, HP), lambda i: (0, 0)),
        ],
        out_specs=[pl.BlockSpec((BN, FC), lambda i: (i, 0))
                   for _ in range(NCH)],
        out_shape=tuple(jax.ShapeDtypeStruct((NP, FC), _f32)
                        for _ in range(NCH)),
    )(x_pad, w1p)


def _tc_scale1(m_chunks, degp):
    def body(*refs):
        m_refs = refs[:NCH]
        d0_ref, d1_ref = refs[NCH:NCH + 2]
        hrefs, dinv_ref = refs[NCH + 2:2 * NCH + 2], refs[2 * NCH + 2]
        i = pl.program_id(0)
        deg = d0_ref[:, 0] + d1_ref[:, 0] + 1.0
        rows = i * BN + lax.broadcasted_iota(jnp.int32, (BN,), 0)
        dinv = jnp.where(rows < N, lax.rsqrt(deg), 0.0)
        dinv_ref[...] = dinv
        for c in range(NCH):
            hrefs[c][...] = m_refs[c][...] * dinv[:, None]

    outs = pl.pallas_call(
        body,
        grid=(GRID_N,),
        in_specs=[pl.BlockSpec((BN, FC), lambda i: (i, 0))
                  for _ in range(NCH)]
        + [
            pl.BlockSpec((BN, FC), lambda i: (i, 0)),
            pl.BlockSpec((BN, FC), lambda i: (i + GRID_N, 0)),
        ],
        out_specs=[pl.BlockSpec((BN, FC), lambda i: (i, 0))
                   for _ in range(NCH)]
        + [pl.BlockSpec((BN,), lambda i: (i,))],
        out_shape=tuple(jax.ShapeDtypeStruct((NP, FC), _f32)
                        for _ in range(NCH))
        + (jax.ShapeDtypeStruct((NP,), _f32),),
    )(*m_chunks, degp, degp)
    return outs[:NCH], outs[NCH]


def _tc_mid(s_parts, h_chunks, dinv, bp, wp):
    def body(*refs):
        s_refs = refs[:2 * NCH]
        h_refs = refs[2 * NCH:3 * NCH]
        dinv_ref, b_ref, w_ref = refs[3 * NCH:3 * NCH + 3]
        orefs = refs[3 * NCH + 3:]
        dinv = dinv_ref[...]
        acc = jnp.zeros((BN, HP), _f32)
        for c in range(NCH):
            s = s_refs[2 * c][...] + s_refs[2 * c + 1][...] + h_refs[c][...]
            hh = jnp.maximum(
                s * dinv[:, None] + b_ref[0, c * FC:(c + 1) * FC], 0.0)
            acc += jnp.dot(hh, w_ref[c * FC:(c + 1) * FC, :],
                           preferred_element_type=_f32)
        hp = acc * dinv[:, None]
        for c, ref in enumerate(orefs):
            ref[...] = hp[:, c * FC:(c + 1) * FC]

    ins = []
    for p in s_parts:
        ins.append(p)
        ins.append(p)
    outs = pl.pallas_call(
        body,
        grid=(GRID_N,),
        in_specs=_chunk_in_specs()
        + [pl.BlockSpec((BN, FC), lambda i: (i, 0)) for _ in range(NCH)]
        + [
            pl.BlockSpec((BN,), lambda i: (i,)),
            pl.BlockSpec((1, HP), lambda i: (0, 0)),
            pl.BlockSpec((HP, HP), lambda i: (0, 0)),
        ],
        out_specs=[pl.BlockSpec((BN, FC), lambda i: (i, 0))
                   for _ in range(NCH)],
        out_shape=tuple(jax.ShapeDtypeStruct((NP, FC), _f32)
                        for _ in range(NCH)),
    )(*ins, *h_chunks, dinv, bp, wp)
    return list(outs)


def _tc_final(s_parts, h_chunks, dinv, bp, batch_pad):
    def body(*refs):
        s_refs = refs[:2 * NCH]
        h_refs = refs[2 * NCH:3 * NCH]
        dinv_ref, b_ref, batch_ref, ne_ref, pool_ref, cnt_ref = \
            refs[3 * NCH:]
        i = pl.program_id(0)
        dinv = dinv_ref[...]
        bb = batch_ref[...]
        onehot = (lax.broadcasted_iota(jnp.int32, (B, BN), 0)
                  == bb[None, :]).astype(_f32)

        @pl.when(i == 0)
        def _():
            pool_ref[...] = jnp.zeros((B, H), _f32)
            cnt_ref[...] = jnp.zeros((B, 128), _f32)

        cnt_ref[...] += jnp.sum(onehot, axis=1)[:, None]

        for c in range(NCH):
            lo = c * FC
            w = min(H - lo, FC)
            if w <= 0:
                continue
            s = s_refs[2 * c][...] + s_refs[2 * c + 1][...] + h_refs[c][...]
            ne_c = (s * dinv[:, None] + b_ref[0, lo:lo + FC])[:, :w]
            ne_ref[:, lo:lo + w] = ne_c
            pool_ref[:, lo:lo + w] += jnp.dot(
                onehot, ne_c, preferred_element_type=_f32)

        @pl.when(i == GRID_N - 1)
        def _():
            pool_ref[...] = pool_ref[...] / jnp.maximum(cnt_ref[:, :1], 1.0)

    ins = []
    for p in s_parts:
        ins.append(p)
        ins.append(p)
    ne, pool = pl.pallas_call(
        body,
        grid=(GRID_N,),
        in_specs=_chunk_in_specs()
        + [pl.BlockSpec((BN, FC), lambda i: (i, 0)) for _ in range(NCH)]
        + [
            pl.BlockSpec((BN,), lambda i: (i,)),
            pl.BlockSpec((1, HP), lambda i: (0, 0)),
            pl.BlockSpec((BN,), lambda i: (i,)),
        ],
        out_specs=[
            pl.BlockSpec((BN, H), lambda i: (i, 0)),
            pl.BlockSpec((B, H), lambda i: (0, 0)),
        ],
        out_shape=(
            jax.ShapeDtypeStruct((N, H), _f32),
            jax.ShapeDtypeStruct((B, H), _f32),
        ),
        scratch_shapes=[pltpu.VMEM((B, 128), _f32)],
    )(*ins, *h_chunks, dinv, bp, batch_pad)
    return ne, pool


def kernel(x, edge_index, batch, W1, b1, W2, b2, W3, b3):
    x_pad = jnp.zeros((NP, D_IN), _f32).at[:N].set(x)
    pad_idx = N + jnp.arange(EP - E, dtype=jnp.int32) % (NP - N)
    src2d = jnp.concatenate([edge_index[0], pad_idx]).reshape(EB, 128)
    dst2d = jnp.concatenate([edge_index[1], pad_idx]).reshape(EB, 128)
    batch_pad = jnp.concatenate(
        [batch.astype(jnp.int32), jnp.full((NP - N,), B, jnp.int32)])

    w1p = jnp.zeros((D_IN, HP), _f32).at[:, :H].set(W1)
    w2p = jnp.zeros((HP, HP), _f32).at[:H, :H].set(W2)
    w3p = jnp.zeros((HP, HP), _f32).at[:H, :H].set(W3)
    b1p = jnp.zeros((1, HP), _f32).at[0, :H].set(b1)
    b2p = jnp.zeros((1, HP), _f32).at[0, :H].set(b2)
    b3p = jnp.zeros((1, HP), _f32).at[0, :H].set(b3)

    degp = _sc_degree(dst2d)
    m1c = _tc_matmul1(x_pad, w1p)
    h1c, dinv = _tc_scale1(m1c, degp)
    s1p = _sc_propagate(src2d, dst2d, h1c)
    h2c = _tc_mid(s1p, h1c, dinv, b1p, w2p)
    s2p = _sc_propagate(src2d, dst2d, h2c)
    h3c = _tc_mid(s2p, h2c, dinv, b2p, w3p)
    s3p = _sc_propagate(src2d, dst2d, h3c)
    ne, pool = _tc_final(s3p, h3c, dinv, b3p, batch_pad)
    return ne, pool


# speedup vs baseline: 1.5528x; 1.5528x over previous
import functools

import jax
import jax.numpy as jnp
from jax import lax
from jax.experimental import pallas as pl
from jax.experimental.pallas import tpu as pltpu
from jax.experimental.pallas import tpu_sc as plsc

N = 10000
NP = 10240
E = 320000
EP = 327680
EB = EP // 128
D_IN = 128
H = 528
HP = 640
FC = 128
NCH = 5
B = 8
BN = 512
GRID_N = NP // BN
TILES = 16
RPT = NP // TILES
WBLK = EB // 32
_f32 = jnp.float32


def _sc_mesh():
    return plsc.VectorSubcoreMesh(core_axis_name="c", subcore_axis_name="s")


def _zero_fill(zeros_v, acc, sid):
    nz = zeros_v.shape[0]

    @pl.loop(0, RPT // nz)
    def _(r):
        pltpu.sync_copy(zeros_v, acc.at[pl.ds(sid * RPT + r * nz, nz)])


def _fill_zeros_buf(zeros_v):
    nr, nc = zeros_v.shape

    @pl.loop(0, nr)
    def _(r):
        @pl.loop(0, nc // 16)
        def _(q):
            zeros_v[r, pl.ds(q * 16, 16)] = jnp.zeros((16,), _f32)


def _sc_degree(dst2d):
    @functools.partial(
        pl.kernel,
        mesh=_sc_mesh(),
        out_type=jax.ShapeDtypeStruct((2 * NP, FC), _f32),
        scratch_types=[
            pltpu.VMEM((WBLK, 128), jnp.int32),
            pltpu.VMEM((128, FC), _f32),
            pltpu.VMEM((64, FC), _f32),
            pltpu.VMEM_SHARED((NP, FC), _f32),
        ],
    )
    def deg_kernel(dst_hbm, out_hbm, idx_v, ones_v, zeros_v, acc):
        cid = lax.axis_index("c")
        sid = lax.axis_index("s")
        wid = cid * TILES + sid

        @pl.loop(0, 128)
        def _(r):
            @pl.loop(0, FC // 16)
            def _(q):
                ones_v[r, pl.ds(q * 16, 16)] = jnp.ones((16,), _f32)

        @pl.loop(0, 64)
        def _(r):
            @pl.loop(0, FC // 16)
            def _(q):
                zeros_v[r, pl.ds(q * 16, 16)] = jnp.zeros((16,), _f32)

        _zero_fill(zeros_v, acc, sid)
        plsc.subcore_barrier()

        pltpu.sync_copy(dst_hbm.at[pl.ds(wid * WBLK, WBLK)], idx_v)

        @pl.loop(0, WBLK)
        def _(j):
            pltpu.sync_copy(ones_v, acc.at[idx_v.at[j]], add=True)

        plsc.subcore_barrier()
        pltpu.sync_copy(
            acc.at[pl.ds(sid * RPT, RPT)],
            out_hbm.at[pl.ds(cid * NP + sid * RPT, RPT)],
        )

    return deg_kernel(dst2d)


def _sc_propagate(src2d, dst2d, hs):
    out_t = tuple(jax.ShapeDtypeStruct((2 * NP, FC), _f32)
                  for _ in range(NCH))

    NBUF = 2
    INFL = NBUF - 1
    BR = 128
    RPW = EP // BR // 32
    NST = 2
    SB = RPW // NST

    @functools.partial(
        pl.kernel,
        mesh=_sc_mesh(),
        out_type=out_t,
        scratch_types=[
            pltpu.VMEM((SB, BR), jnp.int32),
            pltpu.VMEM((SB, BR), jnp.int32),
        ]
        + [pltpu.VMEM((BR, FC), _f32) for _ in range(NBUF)]
        + [pltpu.SemaphoreType.DMA for _ in range(NBUF)]
        + [pltpu.VMEM_SHARED((NP, FC), _f32)],
    )
    def prop_kernel(src_hbm, dst_hbm, h0, h1, h2, h3, h4,
                    s0, s1, s2, s3, s4, sidx_v, didx_v, *rest):
        bufs = rest[:NBUF]
        sems = rest[NBUF:2 * NBUF]
        acc = rest[2 * NBUF]
        cid = lax.axis_index("c")
        sid = lax.axis_index("s")
        wid = cid * TILES + sid

        def run_chunk(h_hbm, s_hbm):
            with jax.named_scope("zero_fill"):
                _fill_zeros_buf(bufs[0])

                @pl.loop(0, RPT // BR)
                def _(r):
                    pltpu.sync_copy(bufs[0],
                                    acc.at[pl.ds(sid * RPT + r * BR, BR)])

                plsc.subcore_barrier()

            for st in range(NST):
                with jax.named_scope("idx_load"):
                    base = wid * RPW + st * SB
                    pltpu.sync_copy(src_hbm.at[pl.ds(base, SB)], sidx_v)
                    pltpu.sync_copy(dst_hbm.at[pl.ds(base, SB)], didx_v)

                with jax.named_scope("edge_pipeline"):
                    for b in range(INFL):
                        pltpu.async_copy(h_hbm.at[sidx_v.at[b]],
                                         bufs[b], sems[b])

                    @pl.loop(0, SB // NBUF)
                    def _(q):
                        j0 = NBUF * q
                        for b in range(NBUF):
                            j = j0 + b
                            jn = lax.min(j + INFL, SB - 1)
                            bi = (b + INFL) % NBUF
                            pltpu.async_copy(h_hbm.at[sidx_v.at[jn]],
                                             bufs[bi], sems[bi])
                            pltpu.make_async_copy(
                                h_hbm.at[sidx_v.at[j]],
                                bufs[b], sems[b]).wait()
                            pltpu.sync_copy(bufs[b], acc.at[didx_v.at[j]],
                                            add=True)

                    for b in range(INFL):
                        pltpu.make_async_copy(
                            h_hbm.at[sidx_v.at[SB - 1]],
                            bufs[b], sems[b]).wait()

            with jax.named_scope("writeback"):
                plsc.subcore_barrier()
                pltpu.sync_copy(acc.at[pl.ds(sid * RPT, RPT)],
                                s_hbm.at[pl.ds(cid * NP + sid * RPT, RPT)])
                plsc.subcore_barrier()

        for h_hbm, s_hbm in zip((h0, h1, h2, h3, h4), (s0, s1, s2, s3, s4)):
            run_chunk(h_hbm, s_hbm)

    return prop_kernel(src2d, dst2d, *hs)


def _chunk_in_specs():
    specs = []
    for _ in range(NCH):
        specs.append(pl.BlockSpec((BN, FC), lambda i: (i, 0)))
        specs.append(pl.BlockSpec((BN, FC), lambda i: (i + GRID_N, 0)))
    return specs


def _tc_matmul1(x_pad, w1p):
    def body(x_ref, w_ref, *orefs):
        m = jnp.dot(x_ref[...], w_ref[...], preferred_element_type=_f32)
        for c, ref in enumerate(orefs):
            ref[...] = m[:, c * FC:(c + 1) * FC]

    return pl.pallas_call(
        body,
        grid=(GRID_N,),
        in_specs=[
            pl.BlockSpec((BN, D_IN), lambda i: (i, 0)),
            pl.BlockSpec((D_IN, HP), lambda i: (0, 0)),
        ],
        out_specs=[pl.BlockSpec((BN, FC), lambda i: (i, 0))
                   for _ in range(NCH)],
        out_shape=tuple(jax.ShapeDtypeStruct((NP, FC), _f32)
                        for _ in range(NCH)),
    )(x_pad, w1p)


def _tc_scale1(m_chunks, degp):
    def body(*refs):
        m_refs = refs[:NCH]
        d0_ref, d1_ref = refs[NCH:NCH + 2]
        hrefs, dinv_ref = refs[NCH + 2:2 * NCH + 2], refs[2 * NCH + 2]
        i = pl.program_id(0)
        deg = d0_ref[:, 0] + d1_ref[:, 0] + 1.0
        rows = i * BN + lax.broadcasted_iota(jnp.int32, (BN,), 0)
        dinv = jnp.where(rows < N, lax.rsqrt(deg), 0.0)
        dinv_ref[...] = dinv
        for c in range(NCH):
            hrefs[c][...] = m_refs[c][...] * dinv[:, None]

    outs = pl.pallas_call(
        body,
        grid=(GRID_N,),
        in_specs=[pl.BlockSpec((BN, FC), lambda i: (i, 0))
                  for _ in range(NCH)]
        + [
            pl.BlockSpec((BN, FC), lambda i: (i, 0)),
            pl.BlockSpec((BN, FC), lambda i: (i + GRID_N, 0)),
        ],
        out_specs=[pl.BlockSpec((BN, FC), lambda i: (i, 0))
                   for _ in range(NCH)]
        + [pl.BlockSpec((BN,), lambda i: (i,))],
        out_shape=tuple(jax.ShapeDtypeStruct((NP, FC), _f32)
                        for _ in range(NCH))
        + (jax.ShapeDtypeStruct((NP,), _f32),),
    )(*m_chunks, degp, degp)
    return outs[:NCH], outs[NCH]


def _tc_mid(s_parts, h_chunks, dinv, bp, wp):
    def body(*refs):
        s_refs = refs[:2 * NCH]
        h_refs = refs[2 * NCH:3 * NCH]
        dinv_ref, b_ref, w_ref = refs[3 * NCH:3 * NCH + 3]
        orefs = refs[3 * NCH + 3:]
        dinv = dinv_ref[...]
        acc = jnp.zeros((BN, HP), _f32)
        for c in range(NCH):
            s = s_refs[2 * c][...] + s_refs[2 * c + 1][...] + h_refs[c][...]
            hh = jnp.maximum(
                s * dinv[:, None] + b_ref[0, c * FC:(c + 1) * FC], 0.0)
            acc += jnp.dot(hh, w_ref[c * FC:(c + 1) * FC, :],
                           preferred_element_type=_f32)
        hp = acc * dinv[:, None]
        for c, ref in enumerate(orefs):
            ref[...] = hp[:, c * FC:(c + 1) * FC]

    ins = []
    for p in s_parts:
        ins.append(p)
        ins.append(p)
    outs = pl.pallas_call(
        body,
        grid=(GRID_N,),
        in_specs=_chunk_in_specs()
        + [pl.BlockSpec((BN, FC), lambda i: (i, 0)) for _ in range(NCH)]
        + [
            pl.BlockSpec((BN,), lambda i: (i,)),
            pl.BlockSpec((1, HP), lambda i: (0, 0)),
            pl.BlockSpec((HP, HP), lambda i: (0, 0)),
        ],
        out_specs=[pl.BlockSpec((BN, FC), lambda i: (i, 0))
                   for _ in range(NCH)],
        out_shape=tuple(jax.ShapeDtypeStruct((NP, FC), _f32)
                        for _ in range(NCH)),
    )(*ins, *h_chunks, dinv, bp, wp)
    return list(outs)


def _tc_final(s_parts, h_chunks, dinv, bp, batch_pad):
    def body(*refs):
        s_refs = refs[:2 * NCH]
        h_refs = refs[2 * NCH:3 * NCH]
        dinv_ref, b_ref, batch_ref, ne_ref, pool_ref, cnt_ref = \
            refs[3 * NCH:]
        i = pl.program_id(0)
        dinv = dinv_ref[...]
        bb = batch_ref[...]
        onehot = (lax.broadcasted_iota(jnp.int32, (B, BN), 0)
                  == bb[None, :]).astype(_f32)

        @pl.when(i == 0)
        def _():
            pool_ref[...] = jnp.zeros((B, H), _f32)
            cnt_ref[...] = jnp.zeros((B, 128), _f32)

        cnt_ref[...] += jnp.sum(onehot, axis=1)[:, None]

        for c in range(NCH):
            lo = c * FC
            w = min(H - lo, FC)
            if w <= 0:
                continue
            s = s_refs[2 * c][...] + s_refs[2 * c + 1][...] + h_refs[c][...]
            ne_c = (s * dinv[:, None] + b_ref[0, lo:lo + FC])[:, :w]
            ne_ref[:, lo:lo + w] = ne_c
            pool_ref[:, lo:lo + w] += jnp.dot(
                onehot, ne_c, preferred_element_type=_f32)

        @pl.when(i == GRID_N - 1)
        def _():
            pool_ref[...] = pool_ref[...] / jnp.maximum(cnt_ref[:, :1], 1.0)

    ins = []
    for p in s_parts:
        ins.append(p)
        ins.append(p)
    ne, pool = pl.pallas_call(
        body,
        grid=(GRID_N,),
        in_specs=_chunk_in_specs()
        + [pl.BlockSpec((BN, FC), lambda i: (i, 0)) for _ in range(NCH)]
        + [
            pl.BlockSpec((BN,), lambda i: (i,)),
            pl.BlockSpec((1, HP), lambda i: (0, 0)),
            pl.BlockSpec((BN,), lambda i: (i,)),
        ],
        out_specs=[
            pl.BlockSpec((BN, H), lambda i: (i, 0)),
            pl.BlockSpec((B, H), lambda i: (0, 0)),
        ],
        out_shape=(
            jax.ShapeDtypeStruct((N, H), _f32),
            jax.ShapeDtypeStruct((B, H), _f32),
        ),
        scratch_shapes=[pltpu.VMEM((B, 128), _f32)],
    )(*ins, *h_chunks, dinv, bp, batch_pad)
    return ne, pool


def kernel(x, edge_index, batch, W1, b1, W2, b2, W3, b3):
    x_pad = jnp.zeros((NP, D_IN), _f32).at[:N].set(x)
    pad_idx = N + jnp.arange(EP - E, dtype=jnp.int32) % (NP - N)
    src2d = jnp.concatenate([edge_index[0], pad_idx]).reshape(EB, 128)
    dst2d = jnp.concatenate([edge_index[1], pad_idx]).reshape(EB, 128)
    batch_pad = jnp.concatenate(
        [batch.astype(jnp.int32), jnp.full((NP - N,), B, jnp.int32)])

    w1p = jnp.zeros((D_IN, HP), _f32).at[:, :H].set(W1)
    w2p = jnp.zeros((HP, HP), _f32).at[:H, :H].set(W2)
    w3p = jnp.zeros((HP, HP), _f32).at[:H, :H].set(W3)
    b1p = jnp.zeros((1, HP), _f32).at[0, :H].set(b1)
    b2p = jnp.zeros((1, HP), _f32).at[0, :H].set(b2)
    b3p = jnp.zeros((1, HP), _f32).at[0, :H].set(b3)

    degp = _sc_degree(dst2d)
    m1c = _tc_matmul1(x_pad, w1p)
    h1c, dinv = _tc_scale1(m1c, degp)
    s1p = _sc_propagate(src2d, dst2d, h1c)
    h2c = _tc_mid(s1p, h1c, dinv, b1p, w2p)
    s2p = _sc_propagate(src2d, dst2d, h2c)
    h3c = _tc_mid(s2p, h2c, dinv, b2p, w3p)
    s3p = _sc_propagate(src2d, dst2d, h3c)
    ne, pool = _tc_final(s3p, h3c, dinv, b3p, batch_pad)
    return ne, pool
